# Initial kernel scaffold; baseline (speedup 1.0000x reference)
import jax
import jax.numpy as jnp
from jax.experimental import pallas as pl


def kernel(x, edge_index, batch, W_l, b_l, W_r, W1, b1, W2, b2):
    raise NotImplementedError("write your pallas kernel here")


# speedup vs baseline: 10.6246x; 10.6246x over previous
import functools

import jax
import jax.numpy as jnp
from jax import lax
from jax.experimental import pallas as pl
from jax.experimental.pallas import tpu as pltpu
from jax.experimental.pallas import tpu_sc as plsc

NCORE = 2
NTILE = 16
CHUNK = 128
DEGW = 16


def _norm_body(x_ref, xn_ref, mu_ref, rs_ref):
    xb = x_ref[0]
    mu = jnp.mean(xb, axis=0, keepdims=True)
    xc = xb - mu
    var = jnp.mean(xc * xc, axis=0, keepdims=True)
    rstd = lax.rsqrt(var + 1e-5)
    xn_ref[0] = xc * rstd
    mu_ref[0] = mu
    rs_ref[0] = rstd


def _instance_norm(x):
    B, N, C = x.shape
    return pl.pallas_call(
        _norm_body,
        grid=(B,),
        in_specs=[pl.BlockSpec((1, N, C), lambda b: (b, 0, 0))],
        out_specs=[
            pl.BlockSpec((1, N, C), lambda b: (b, 0, 0)),
            pl.BlockSpec((1, 1, C), lambda b: (b, 0, 0)),
            pl.BlockSpec((1, 1, C), lambda b: (b, 0, 0)),
        ],
        out_shape=[
            jax.ShapeDtypeStruct((B, N, C), jnp.float32),
            jax.ShapeDtypeStruct((B, 1, C), jnp.float32),
            jax.ShapeDtypeStruct((B, 1, C), jnp.float32),
        ],
    )(x)


def _sc_aggregate(xn2, srcs, dsts, z128, ones128, N, C, BN):
    B = BN // N
    EC = srcs.shape[0] // B
    cpt = EC // NTILE
    cpt_deg = cpt // NCORE
    NH = N // 2
    npad = -(-(NH + 104) // 1280) * 1280
    rzt = npad // NTILE
    ZBR = 80
    QR = 8
    assert rzt % ZBR == 0 and cpt % QR == 0 and cpt_deg % QR == 0
    bpc = B // NCORE
    nslab = 2 * B + 2 * NCORE
    wb_sizes = []
    left = rzt
    while left > 0:
        wb_sizes.append(min(left, CHUNK))
        left -= wb_sizes[-1]

    mesh = plsc.VectorSubcoreMesh(core_axis_name="c", subcore_axis_name="s")

    @functools.partial(
        pl.kernel,
        out_type=jax.ShapeDtypeStruct((nslab * npad, C), jnp.float32),
        mesh=mesh,
        scratch_types=[
            pltpu.VMEM((QR, CHUNK), jnp.int32),
            pltpu.VMEM((QR, CHUNK), jnp.int32),
            pltpu.VMEM((CHUNK, C), jnp.float32),
            pltpu.VMEM_SHARED((npad, C), jnp.float32),
            pltpu.SemaphoreType.DMA,
        ],
    )
    def sc_kernel(xn_hbm, src_hbm, dst_hbm, z128_hbm, ones_hbm, agg_hbm,
                  src_blk, dst_blk, rows_v, acc, sem):
        c = lax.axis_index("c")
        s = lax.axis_index("s")

        def zero_acc():
            pltpu.sync_copy(z128_hbm, rows_v.at[pl.ds(0, ZBR)])
            for j in range(rzt // ZBR):
                pltpu.sync_copy(rows_v.at[pl.ds(0, ZBR)],
                                acc.at[pl.ds(s * rzt + j * ZBR, ZBR)])

        def writeback(slab):
            obase = slab * npad + s * rzt
            roff = 0
            for sz in wb_sizes:
                pltpu.sync_copy(acc.at[pl.ds(s * rzt + roff, sz)],
                                rows_v.at[pl.ds(0, sz)])
                pltpu.sync_copy(rows_v.at[pl.ds(0, sz)],
                                agg_hbm.at[pl.ds(obase + roff, sz)])
                roff += sz

        for p in range(bpc):
            b = c * bpc + p
            for hn in range(2):
                zero_acc()
                plsc.subcore_barrier()

                @pl.loop(0, cpt // QR)
                def _(q):
                    i0 = s * cpt + q * QR
                    pltpu.sync_copy(
                        src_hbm.at[pl.ds(b * EC + i0, QR)], src_blk)
                    pltpu.sync_copy(
                        dst_hbm.at[pl.ds(hn * EC + i0, QR)], dst_blk)
                    for k in range(QR):
                        pltpu.async_copy(
                            xn_hbm.at[src_blk.at[k]], rows_v, sem).wait()
                        pltpu.sync_copy(rows_v, acc.at[dst_blk.at[k]],
                                        add=True)

                plsc.subcore_barrier()
                writeback(2 * b + hn)
                plsc.subcore_barrier()

        for hn in range(2):
            zero_acc()
            pltpu.sync_copy(ones_hbm, rows_v)
            plsc.subcore_barrier()

            @pl.loop(0, cpt_deg // QR)
            def _(q):
                i0 = hn * EC + c * (EC // NCORE) + s * cpt_deg + q * QR
                pltpu.sync_copy(dst_hbm.at[pl.ds(i0, QR)], dst_blk)
                for k in range(QR):
                    pltpu.sync_copy(rows_v, acc.at[dst_blk.at[k]],
                                    add=True)

            plsc.subcore_barrier()
            writeback(2 * B + 2 * c + hn)
            plsc.subcore_barrier()

    return sc_kernel(xn2, srcs, dsts, z128, ones128)


def _post_body(x_ref, mu_ref, rs_ref, agg_ref, d0_ref, d1_ref,
               wl_ref, bl_ref, wr_ref, w1_ref, b1_ref, w2_ref, b2_ref,
               o_ref):
    xb = x_ref[0]
    xn = (xb - mu_ref[0]) * rs_ref[0]
    deg = jnp.maximum(d0_ref[0, :, 0] + d1_ref[0, :, 0], 1.0)
    agg = agg_ref[0] * (1.0 / deg)[:, None]
    dot = functools.partial(lax.dot, precision=lax.Precision.HIGHEST,
                            preferred_element_type=jnp.float32)
    h = dot(agg, wl_ref[...]) + bl_ref[...] + dot(xn, wr_ref[...])
    t = jnp.maximum(dot(h, w1_ref[...]) + b1_ref[...], 0.0)
    a = jax.nn.sigmoid(dot(t, w2_ref[...]) + b2_ref[...])
    o_ref[0] = jnp.maximum(a * h + xb, 0.0)


def _post(x, mu, rs, aggp, wl_t, b_l, wr_t, w1_t, b1, w2_t, b2, blk):
    B, N, C = x.shape
    R = w1_t.shape[1]
    nb = N // blk
    hb = (N // 2) // blk
    full = lambda *shape: pl.BlockSpec(shape, lambda b, i: (0,) * len(shape))
    return pl.pallas_call(
        _post_body,
        grid=(B, nb),
        in_specs=[
            pl.BlockSpec((1, blk, C), lambda b, i: (b, i, 0)),
            pl.BlockSpec((1, 1, C), lambda b, i: (b, 0, 0)),
            pl.BlockSpec((1, 1, C), lambda b, i: (b, 0, 0)),
            pl.BlockSpec((1, blk, C),
                         lambda b, i: (2 * b + i // hb, i % hb, 0)),
            pl.BlockSpec((1, blk, C),
                         lambda b, i: (2 * B + i // hb, i % hb, 0)),
            pl.BlockSpec((1, blk, C),
                         lambda b, i: (2 * B + 2 + i // hb, i % hb, 0)),
            full(C, C),
            full(1, C),
            full(C, C),
            full(C, R),
            full(1, R),
            full(R, C),
            full(1, C),
        ],
        out_specs=[pl.BlockSpec((1, blk, C), lambda b, i: (b, i, 0))],
        out_shape=[jax.ShapeDtypeStruct((B, N, C), jnp.float32)],
    )(x, mu, rs, aggp, aggp, aggp, wl_t, b_l, wr_t, w1_t, b1, w2_t, b2)[0]


def kernel(x, edge_index, batch, W_l, b_l, W_r, W1, b1, W2, b2):
    B, N, C = x.shape
    E = edge_index.shape[1]
    NH = N // 2

    align = NCORE * 8
    cpt = -(-E // (NTILE * CHUNK))
    cpt = -(-cpt // align) * align
    E_pad = NTILE * CHUNK * cpt
    EC = E_pad // CHUNK
    pad = E_pad - E
    src = jnp.concatenate([edge_index[0], jnp.zeros((pad,), jnp.int32)])
    ar = jnp.arange(E_pad, dtype=jnp.int32)
    dst = jnp.concatenate([edge_index[1], N + (ar[:pad] % 96)])

    srcs = src[None, :] + (jnp.arange(B, dtype=jnp.int32) * N)[:, None]
    srcs = srcs.reshape(B * EC, CHUNK)
    t = dst[None, :] - (jnp.arange(2, dtype=jnp.int32) * NH)[:, None]
    trash = NH + 8 + (ar % 96)
    dsts = jnp.where((t >= 0) & (t < NH), t, trash[None, :])
    dsts = dsts.reshape(2 * EC, CHUNK)

    z128 = jnp.zeros((80, C), jnp.float32)
    ones128 = jnp.ones((CHUNK, C), jnp.float32)

    xn, mu, rs = _instance_norm(x)
    aggp = _sc_aggregate(xn.reshape(B * N, C), srcs, dsts,
                         z128, ones128, N, C, B * N)

    npad = -(-(NH + 104) // 1280) * 1280
    out = _post(x, mu, rs, aggp.reshape(2 * B + 4, npad, C),
                W_l.T, b_l.reshape(1, C), W_r.T, W1.T, b1.reshape(1, -1),
                W2.T, b2.reshape(1, C), blk=1000)
    return out


# speedup vs baseline: 12.3434x; 1.1618x over previous
import functools

import jax
import jax.numpy as jnp
from jax import lax
from jax.experimental import pallas as pl
from jax.experimental.pallas import tpu as pltpu
from jax.experimental.pallas import tpu_sc as plsc

NCORE = 2
NTILE = 16
CHUNK = 128
DEGW = 16


def _norm_body(x_ref, xn_ref, mu_ref, rs_ref):
    xb = x_ref[0]
    mu = jnp.mean(xb, axis=0, keepdims=True)
    xc = xb - mu
    var = jnp.mean(xc * xc, axis=0, keepdims=True)
    rstd = lax.rsqrt(var + 1e-5)
    xn_ref[0] = xc * rstd
    mu_ref[0] = mu
    rs_ref[0] = rstd


def _instance_norm(x):
    B, N, C = x.shape
    return pl.pallas_call(
        _norm_body,
        grid=(B,),
        in_specs=[pl.BlockSpec((1, N, C), lambda b: (b, 0, 0))],
        out_specs=[
            pl.BlockSpec((1, N, C), lambda b: (b, 0, 0)),
            pl.BlockSpec((1, 1, C), lambda b: (b, 0, 0)),
            pl.BlockSpec((1, 1, C), lambda b: (b, 0, 0)),
        ],
        out_shape=[
            jax.ShapeDtypeStruct((B, N, C), jnp.float32),
            jax.ShapeDtypeStruct((B, 1, C), jnp.float32),
            jax.ShapeDtypeStruct((B, 1, C), jnp.float32),
        ],
    )(x)


def _sc_aggregate(xn2, srcs, dsts, z128, ones128, N, C, BN):
    B = BN // N
    EC = srcs.shape[0] // B
    cpt = EC // NTILE
    cpt_deg = cpt // NCORE
    NH = N // 2
    npad = -(-(NH + 104) // 1280) * 1280
    rzt = npad // NTILE
    ZBR = 80
    QR = 8
    assert rzt % ZBR == 0 and cpt % QR == 0 and cpt_deg % QR == 0
    bpc = B // NCORE
    nslab = 2 * B + 2 * NCORE
    wb_sizes = []
    left = rzt
    while left > 0:
        wb_sizes.append(min(left, CHUNK))
        left -= wb_sizes[-1]

    mesh = plsc.VectorSubcoreMesh(core_axis_name="c", subcore_axis_name="s")

    @functools.partial(
        pl.kernel,
        out_type=jax.ShapeDtypeStruct((nslab * npad, C), jnp.float32),
        mesh=mesh,
        scratch_types=[
            pltpu.VMEM((QR, CHUNK), jnp.int32),
            pltpu.VMEM((QR, CHUNK), jnp.int32),
            pltpu.VMEM((CHUNK, C), jnp.float32),
            pltpu.VMEM((CHUNK, C), jnp.float32),
            pltpu.VMEM((CHUNK, C), jnp.float32),
            pltpu.VMEM_SHARED((npad, C), jnp.float32),
            pltpu.SemaphoreType.DMA,
            pltpu.SemaphoreType.DMA,
            pltpu.SemaphoreType.DMA,
            pltpu.SemaphoreType.DMA,
        ],
    )
    def sc_kernel(xn_hbm, src_hbm, dst_hbm, z128_hbm, ones_hbm, agg_hbm,
                  src_blk, dst_blk, rows_v, rows_w, rows_u, acc,
                  sem, sem2, sem3, sem4):
        bufs = (rows_v, rows_w, rows_u)
        sems = (sem, sem2, sem3)
        c = lax.axis_index("c")
        s = lax.axis_index("s")

        def zero_acc():
            pltpu.sync_copy(z128_hbm, rows_v.at[pl.ds(0, ZBR)])
            for j in range(rzt // ZBR):
                pltpu.sync_copy(rows_v.at[pl.ds(0, ZBR)],
                                acc.at[pl.ds(s * rzt + j * ZBR, ZBR)])

        def writeback(slab):
            obase = slab * npad + s * rzt
            roff = 0
            for sz in wb_sizes:
                pltpu.sync_copy(acc.at[pl.ds(s * rzt + roff, sz)],
                                rows_v.at[pl.ds(0, sz)])
                pltpu.sync_copy(rows_v.at[pl.ds(0, sz)],
                                agg_hbm.at[pl.ds(obase + roff, sz)])
                roff += sz

        for p in range(bpc):
            b = c * bpc + p
            for hn in range(2):
                zero_acc()
                plsc.subcore_barrier()

                @pl.loop(0, cpt // QR)
                def _(q):
                    i0 = s * cpt + q * QR
                    pltpu.sync_copy(
                        src_hbm.at[pl.ds(b * EC + i0, QR)], src_blk)
                    pltpu.sync_copy(
                        dst_hbm.at[pl.ds(hn * EC + i0, QR)], dst_blk)
                    cps = [None, None, None]
                    for d in range(2):
                        cps[d] = pltpu.async_copy(
                            xn_hbm.at[src_blk.at[d]], bufs[d], sems[d])
                    for k in range(QR):
                        if k + 2 < QR:
                            j = (k + 2) % 3
                            cps[j] = pltpu.async_copy(
                                xn_hbm.at[src_blk.at[k + 2]], bufs[j],
                                sems[j])
                        cps[k % 3].wait()
                        pltpu.sync_copy(bufs[k % 3],
                                        acc.at[dst_blk.at[k]], add=True)

                plsc.subcore_barrier()
                writeback(2 * b + hn)
                plsc.subcore_barrier()

        for hn in range(2):
            zero_acc()
            pltpu.sync_copy(ones_hbm, rows_v)
            plsc.subcore_barrier()

            @pl.loop(0, cpt_deg // QR)
            def _(q):
                i0 = hn * EC + c * (EC // NCORE) + s * cpt_deg + q * QR
                pltpu.sync_copy(dst_hbm.at[pl.ds(i0, QR)], dst_blk)
                dcps = [
                    pltpu.async_copy(rows_v, acc.at[dst_blk.at[k]],
                                     sem4, add=True)
                    for k in range(QR)
                ]
                for cp in dcps:
                    cp.wait()

            plsc.subcore_barrier()
            writeback(2 * B + 2 * c + hn)
            plsc.subcore_barrier()

    return sc_kernel(xn2, srcs, dsts, z128, ones128)


def _post_body(x_ref, mu_ref, rs_ref, agg_ref, d0_ref, d1_ref,
               wl_ref, bl_ref, wr_ref, w1_ref, b1_ref, w2_ref, b2_ref,
               o_ref):
    xb = x_ref[0]
    xn = (xb - mu_ref[0]) * rs_ref[0]
    deg = jnp.maximum(d0_ref[0, :, 0] + d1_ref[0, :, 0], 1.0)
    agg = agg_ref[0] * (1.0 / deg)[:, None]
    dot = functools.partial(lax.dot, precision=lax.Precision.HIGHEST,
                            preferred_element_type=jnp.float32)
    h = dot(agg, wl_ref[...]) + bl_ref[...] + dot(xn, wr_ref[...])
    t = jnp.maximum(dot(h, w1_ref[...]) + b1_ref[...], 0.0)
    a = jax.nn.sigmoid(dot(t, w2_ref[...]) + b2_ref[...])
    o_ref[0] = jnp.maximum(a * h + xb, 0.0)


def _post(x, mu, rs, aggp, wl_t, b_l, wr_t, w1_t, b1, w2_t, b2, blk):
    B, N, C = x.shape
    R = w1_t.shape[1]
    nb = N // blk
    hb = (N // 2) // blk
    full = lambda *shape: pl.BlockSpec(shape, lambda b, i: (0,) * len(shape))
    return pl.pallas_call(
        _post_body,
        grid=(B, nb),
        in_specs=[
            pl.BlockSpec((1, blk, C), lambda b, i: (b, i, 0)),
            pl.BlockSpec((1, 1, C), lambda b, i: (b, 0, 0)),
            pl.BlockSpec((1, 1, C), lambda b, i: (b, 0, 0)),
            pl.BlockSpec((1, blk, C),
                         lambda b, i: (2 * b + i // hb, i % hb, 0)),
            pl.BlockSpec((1, blk, C),
                         lambda b, i: (2 * B + i // hb, i % hb, 0)),
            pl.BlockSpec((1, blk, C),
                         lambda b, i: (2 * B + 2 + i // hb, i % hb, 0)),
            full(C, C),
            full(1, C),
            full(C, C),
            full(C, R),
            full(1, R),
            full(R, C),
            full(1, C),
        ],
        out_specs=[pl.BlockSpec((1, blk, C), lambda b, i: (b, i, 0))],
        out_shape=[jax.ShapeDtypeStruct((B, N, C), jnp.float32)],
    )(x, mu, rs, aggp, aggp, aggp, wl_t, b_l, wr_t, w1_t, b1, w2_t, b2)[0]


def kernel(x, edge_index, batch, W_l, b_l, W_r, W1, b1, W2, b2):
    B, N, C = x.shape
    E = edge_index.shape[1]
    NH = N // 2

    align = NCORE * 8
    cpt = -(-E // (NTILE * CHUNK))
    cpt = -(-cpt // align) * align
    E_pad = NTILE * CHUNK * cpt
    EC = E_pad // CHUNK
    pad = E_pad - E
    src = jnp.concatenate([edge_index[0], jnp.zeros((pad,), jnp.int32)])
    ar = jnp.arange(E_pad, dtype=jnp.int32)
    dst = jnp.concatenate([edge_index[1], N + (ar[:pad] % 96)])

    srcs = src[None, :] + (jnp.arange(B, dtype=jnp.int32) * N)[:, None]
    srcs = srcs.reshape(B * EC, CHUNK)
    t = dst[None, :] - (jnp.arange(2, dtype=jnp.int32) * NH)[:, None]
    trash = NH + 8 + (ar % 96)
    dsts = jnp.where((t >= 0) & (t < NH), t, trash[None, :])
    dsts = dsts.reshape(2 * EC, CHUNK)

    z128 = jnp.zeros((80, C), jnp.float32)
    ones128 = jnp.ones((CHUNK, C), jnp.float32)

    xn, mu, rs = _instance_norm(x)
    aggp = _sc_aggregate(xn.reshape(B * N, C), srcs, dsts,
                         z128, ones128, N, C, B * N)

    npad = -(-(NH + 104) // 1280) * 1280
    out = _post(x, mu, rs, aggp.reshape(2 * B + 4, npad, C),
                W_l.T, b_l.reshape(1, C), W_r.T, W1.T, b1.reshape(1, -1),
                W2.T, b2.reshape(1, C), blk=1000)
    return out
